# baseline (speedup 1.0000x reference)
import dataclasses
import functools

import jax
import jax.numpy as jnp
from jax import lax
from jax.experimental import pallas as pl
from jax.experimental.pallas import tpu as pltpu
from jax.experimental.pallas import tpu_sc as plsc

NC = 2
NS = 16
LANES = 16
B = 128
ZC = 80
NQ = 4
NB = 3
UN = 12


def _sc_spmm(x, comb, epb):
    n, d = x.shape
    nchunk = n // ZC
    mesh = plsc.VectorSubcoreMesh(core_axis_name="c", subcore_axis_name="s")
    cp = pltpu.CompilerParams()
    if "needs_layout_passes" in pltpu.CompilerParams.__dataclass_fields__:
        cp = dataclasses.replace(cp, needs_layout_passes=False)

    @functools.partial(
        pl.kernel,
        out_type=jax.ShapeDtypeStruct((NC, n, d), jnp.float32),
        mesh=mesh,
        compiler_params=cp,
        scratch_types=[
            pltpu.VMEM((NQ * 3, 1, B), jnp.int32),
            pltpu.VMEM((B, d), jnp.float32),
            pltpu.VMEM((B, d), jnp.float32),
            pltpu.VMEM((B, d), jnp.float32),
            pltpu.VMEM_SHARED((n, d), jnp.float32),
            pltpu.SemaphoreType.DMA,
            pltpu.SemaphoreType.DMA,
            pltpu.SemaphoreType.DMA,
            pltpu.SemaphoreType.DMA,
            pltpu.SemaphoreType.DMA,
            pltpu.SemaphoreType.DMA,
            pltpu.SemaphoreType.DMA,
            pltpu.SemaphoreType.DMA,
            pltpu.SemaphoreType.DMA,
            pltpu.SemaphoreType.DMA,
        ],
    )
    def k(x_hbm, comb_hbm, out_hbm, comb_v, rows0, rows1, rows2, acc_sh,
          si0, si1, si2, si3, sg0, sg1, sg2, ss0, ss1, ss2):
        c = lax.axis_index("c")
        s = lax.axis_index("s")
        wid = s * NC + c
        lo = wid * epb
        isems = (si0, si1, si2, si3)
        bufs = (rows0, rows1, rows2)
        gsems = (sg0, sg1, sg2)
        ssems = (ss0, ss1, ss2)

        def idx_start(t, q):
            pltpu.async_copy(comb_hbm.at[pl.ds((lo + t) * 3, 3)],
                             comb_v.at[pl.ds(q * 3, 3)], isems[q])

        def idx_wait(t, q):
            pltpu.make_async_copy(comb_hbm.at[pl.ds((lo + t) * 3, 3)],
                                  comb_v.at[pl.ds(q * 3, 3)], isems[q]).wait()

        def gather_start(q, p):
            pltpu.async_copy(x_hbm.at[comb_v.at[q * 3, 0]], bufs[p], gsems[p])

        def gather_wait(q, p):
            pltpu.make_async_copy(
                x_hbm.at[comb_v.at[q * 3, 0]], bufs[p], gsems[p]).wait()

        def scale(q, p):
            buf = bufs[p]

            @pl.loop(0, B)
            def _(i):
                wb = plsc.bitcast(
                    plsc.load_gather(
                        comb_v, [jnp.full((LANES,), q * 3 + 2, jnp.int32),
                                 jnp.full((LANES,), 0, jnp.int32),
                                 jnp.full((LANES,), i, jnp.int32)]),
                    jnp.float32)
                for jj in range(d // LANES):
                    sl = pl.ds(jj * LANES, LANES)
                    buf[i, sl] = buf[i, sl] * wb

        def scatter_start(q, p):
            pltpu.async_copy(bufs[p], acc_sh.at[comb_v.at[q * 3 + 1, 0]],
                             ssems[p], add=True)

        def scatter_wait(q, p):
            pltpu.make_async_copy(bufs[p], acc_sh.at[comb_v.at[q * 3 + 1, 0]],
                                  ssems[p]).wait()

        for t in range(3):
            idx_start(t, t)

        zero = jnp.zeros((LANES,), jnp.float32)

        @pl.loop(0, ZC)
        def _(i):
            for j in range(d // LANES):
                rows0[i, pl.ds(j * LANES, LANES)] = zero

        @pl.loop(s, nchunk, step=NS)
        def _(t):
            pltpu.sync_copy(rows0.at[pl.ds(0, ZC)],
                            acc_sh.at[pl.ds(t * ZC, ZC)])

        idx_wait(0, 0)
        gather_start(0, 0)
        idx_wait(1, 1)
        gather_start(1, 1)
        plsc.subcore_barrier()

        @pl.loop(0, epb, step=UN)
        def _(j):
            for b in range(UN):
                t = j + b
                q, p = b % NQ, b % NB
                q2, p2 = (b + 2) % NQ, (b + 2) % NB
                qprev = (b + 3) % NQ

                @pl.when(t + 2 < epb)
                def _():
                    idx_wait(t + 2, q2)

                gather_wait(q, p)
                scale(q, p)

                @pl.when(t > 0)
                def _():
                    scatter_wait(qprev, p2)

                @pl.when(t + 2 < epb)
                def _():
                    gather_start(q2, p2)

                scatter_start(q, p)

                @pl.when(t + 3 < epb)
                def _():
                    idx_start(t + 3, (b + 3) % NQ)

        scatter_wait((epb - 1) % NQ, (epb - 1) % NB)
        plsc.subcore_barrier()

        @pl.loop(s, nchunk, step=NS)
        def _(t):
            pltpu.sync_copy(acc_sh.at[pl.ds(t * ZC, ZC)],
                            out_hbm.at[c].at[pl.ds(t * ZC, ZC)])

    return k(x, comb)


def _tc_finish(y, w):
    _, n, d = y.shape
    blk = 2000

    def body(y_ref, w_ref, o_ref):
        acc = y_ref[0] + y_ref[1]
        o_ref[...] = jnp.maximum(
            jnp.dot(acc, w_ref[...], preferred_element_type=jnp.float32), 0.0)

    return pl.pallas_call(
        body,
        out_shape=jax.ShapeDtypeStruct((n, d), jnp.float32),
        grid=(n // blk,),
        in_specs=[
            pl.BlockSpec((2, blk, d), lambda i: (0, i, 0)),
            pl.BlockSpec((d, d), lambda i: (0, 0)),
        ],
        out_specs=pl.BlockSpec((blk, d), lambda i: (i, 0)),
    )(y, w)


def kernel(x, edge_index, edge_weight, W):
    e = edge_index.shape[1]
    nw = NC * NS
    epb = -(-e // (nw * B))
    epb = -(-epb // UN) * UN
    ep = nw * epb * B
    pad = ep - e
    pad_idx = (jnp.arange(pad, dtype=jnp.int32) %
               jnp.int32(x.shape[0])).astype(jnp.int32)
    src = jnp.concatenate([edge_index[0], pad_idx]).reshape(-1, 1, B)
    dst = jnp.concatenate([edge_index[1], pad_idx]).reshape(-1, 1, B)
    wbits = jax.lax.bitcast_convert_type(
        jnp.concatenate([edge_weight, jnp.zeros((pad,), jnp.float32)]),
        jnp.int32).reshape(-1, 1, B)
    comb = jnp.concatenate([src, dst, wbits],
                           axis=1).reshape(-1, 1, B)
    y = _sc_spmm(x, comb, epb)
    return _tc_finish(y, W)


# speedup vs baseline: 1.1648x; 1.1648x over previous
import dataclasses
import functools

import jax
import jax.numpy as jnp
from jax import lax
from jax.experimental import pallas as pl
from jax.experimental.pallas import tpu as pltpu
from jax.experimental.pallas import tpu_sc as plsc

NC = 2
NS = 16
LANES = 16
B = 128
ZC = 80
NQ = 4


def _sc_spmm(x, comb, epb):
    n, d = x.shape
    nchunk = n // ZC
    mesh = plsc.VectorSubcoreMesh(core_axis_name="c", subcore_axis_name="s")
    cp = pltpu.CompilerParams()
    if "needs_layout_passes" in pltpu.CompilerParams.__dataclass_fields__:
        cp = dataclasses.replace(cp, needs_layout_passes=False)

    @functools.partial(
        pl.kernel,
        out_type=jax.ShapeDtypeStruct((NC, n, d), jnp.float32),
        mesh=mesh,
        compiler_params=cp,
        scratch_types=[
            pltpu.VMEM((NQ * 3, 1, B), jnp.int32),
            pltpu.VMEM((B, d), jnp.float32),
            pltpu.VMEM((B, d), jnp.float32),
            pltpu.VMEM_SHARED((n, d), jnp.float32),
            pltpu.SemaphoreType.DMA,
            pltpu.SemaphoreType.DMA,
            pltpu.SemaphoreType.DMA,
            pltpu.SemaphoreType.DMA,
            pltpu.SemaphoreType.DMA,
            pltpu.SemaphoreType.DMA,
            pltpu.SemaphoreType.DMA,
            pltpu.SemaphoreType.DMA,
        ],
    )
    def k(x_hbm, comb_hbm, out_hbm, comb_v, rows0, rows1, acc_sh,
          si0, si1, si2, si3, sg0, sg1, ss0, ss1):
        c = lax.axis_index("c")
        s = lax.axis_index("s")
        wid = s * NC + c
        lo = wid * epb
        isems = (si0, si1, si2, si3)
        bufs = (rows0, rows1)
        gsems = (sg0, sg1)
        ssems = (ss0, ss1)

        def idx_start(t, q):
            pltpu.async_copy(comb_hbm.at[pl.ds((lo + t) * 3, 3)],
                             comb_v.at[pl.ds(q * 3, 3)], isems[q])

        def idx_wait(t, q):
            pltpu.make_async_copy(comb_hbm.at[pl.ds((lo + t) * 3, 3)],
                                  comb_v.at[pl.ds(q * 3, 3)], isems[q]).wait()

        def gather_start(q, p):
            pltpu.async_copy(x_hbm.at[comb_v.at[q * 3, 0]], bufs[p], gsems[p])

        def gather_wait(q, p):
            pltpu.make_async_copy(
                x_hbm.at[comb_v.at[q * 3, 0]], bufs[p], gsems[p]).wait()

        def scale(q, p):
            buf = bufs[p]

            @pl.loop(0, B, step=2)
            def _(i):
                row = jnp.full((LANES,), q * 3 + 2, jnp.int32)
                col = jnp.full((LANES,), 0, jnp.int32)
                wb0 = plsc.bitcast(
                    plsc.load_gather(
                        comb_v, [row, col,
                                 jnp.full((LANES,), i, jnp.int32)]),
                    jnp.float32)
                wb1 = plsc.bitcast(
                    plsc.load_gather(
                        comb_v, [row, col,
                                 jnp.full((LANES,), i + 1, jnp.int32)]),
                    jnp.float32)
                for jj in range(d // LANES):
                    sl = pl.ds(jj * LANES, LANES)
                    buf[i, sl] = buf[i, sl] * wb0
                for jj in range(d // LANES):
                    sl = pl.ds(jj * LANES, LANES)
                    buf[i + 1, sl] = buf[i + 1, sl] * wb1

        def scatter_start(q, p):
            pltpu.async_copy(bufs[p], acc_sh.at[comb_v.at[q * 3 + 1, 0]],
                             ssems[p], add=True)

        def scatter_wait(q, p):
            pltpu.make_async_copy(bufs[p], acc_sh.at[comb_v.at[q * 3 + 1, 0]],
                                  ssems[p]).wait()

        for t in range(3):
            idx_start(t, t)

        zero = jnp.zeros((LANES,), jnp.float32)

        @pl.loop(0, ZC)
        def _(i):
            for j in range(d // LANES):
                rows0[i, pl.ds(j * LANES, LANES)] = zero

        @pl.loop(s, nchunk, step=NS)
        def _(t):
            pltpu.sync_copy(rows0.at[pl.ds(0, ZC)],
                            acc_sh.at[pl.ds(t * ZC, ZC)])

        idx_wait(0, 0)
        gather_start(0, 0)
        plsc.subcore_barrier()

        @pl.loop(0, epb, step=NQ)
        def _(j):
            for b in range(NQ):
                t = j + b
                q, p = b, b % 2
                qn, pn = (b + 1) % NQ, (b + 1) % 2
                qprev = (b + 3) % NQ

                @pl.when(t + 1 < epb)
                def _():
                    idx_wait(t + 1, qn)

                gather_wait(q, p)

                @pl.when(t > 0)
                def _():
                    scatter_wait(qprev, pn)

                @pl.when(t + 1 < epb)
                def _():
                    gather_start(qn, pn)

                scale(q, p)
                scatter_start(q, p)

                @pl.when(t + 3 < epb)
                def _():
                    idx_start(t + 3, (b + 3) % NQ)

        scatter_wait((epb - 1) % NQ, (epb - 1) % 2)
        plsc.subcore_barrier()

        @pl.loop(s, nchunk, step=NS)
        def _(t):
            pltpu.sync_copy(acc_sh.at[pl.ds(t * ZC, ZC)],
                            out_hbm.at[c].at[pl.ds(t * ZC, ZC)])

    return k(x, comb)


def _tc_finish(y, w):
    _, n, d = y.shape
    blk = 2000

    def body(y_ref, w_ref, o_ref):
        acc = y_ref[0] + y_ref[1]
        o_ref[...] = jnp.maximum(
            jnp.dot(acc, w_ref[...], preferred_element_type=jnp.float32), 0.0)

    return pl.pallas_call(
        body,
        out_shape=jax.ShapeDtypeStruct((n, d), jnp.float32),
        grid=(n // blk,),
        in_specs=[
            pl.BlockSpec((2, blk, d), lambda i: (0, i, 0)),
            pl.BlockSpec((d, d), lambda i: (0, 0)),
        ],
        out_specs=pl.BlockSpec((blk, d), lambda i: (i, 0)),
    )(y, w)


def kernel(x, edge_index, edge_weight, W):
    e = edge_index.shape[1]
    nw = NC * NS
    epb = -(-e // (nw * B))
    epb = -(-epb // NQ) * NQ
    ep = nw * epb * B
    pad = ep - e
    pad_idx = (jnp.arange(pad, dtype=jnp.int32) %
               jnp.int32(x.shape[0])).astype(jnp.int32)
    src = jnp.concatenate([edge_index[0], pad_idx]).reshape(-1, 1, B)
    dst = jnp.concatenate([edge_index[1], pad_idx]).reshape(-1, 1, B)
    wbits = jax.lax.bitcast_convert_type(
        jnp.concatenate([edge_weight, jnp.zeros((pad,), jnp.float32)]),
        jnp.int32).reshape(-1, 1, B)
    comb = jnp.concatenate([src, dst, wbits],
                           axis=1).reshape(-1, 1, B)
    y = _sc_spmm(x, comb, epb)
    return _tc_finish(y, W)


# speedup vs baseline: 1.1996x; 1.0299x over previous
import dataclasses
import functools

import jax
import jax.numpy as jnp
from jax import lax
from jax.experimental import pallas as pl
from jax.experimental.pallas import tpu as pltpu
from jax.experimental.pallas import tpu_sc as plsc

NC = 2
NS = 16
LANES = 16
B = 128
ZC = 80
NQ = 4


def _sc_spmm(x, comb, epb):
    n, d = x.shape
    nchunk = n // ZC
    mesh = plsc.VectorSubcoreMesh(core_axis_name="c", subcore_axis_name="s")
    cp = pltpu.CompilerParams()
    if "needs_layout_passes" in pltpu.CompilerParams.__dataclass_fields__:
        cp = dataclasses.replace(cp, needs_layout_passes=False)

    @functools.partial(
        pl.kernel,
        out_type=jax.ShapeDtypeStruct((NC, n, d), jnp.float32),
        mesh=mesh,
        compiler_params=cp,
        scratch_types=[
            pltpu.VMEM((NQ * 3, 1, B), jnp.int32),
            pltpu.VMEM((B, d), jnp.float32),
            pltpu.VMEM((B, d), jnp.float32),
            pltpu.VMEM_SHARED((n, d), jnp.float32),
            pltpu.SemaphoreType.DMA,
            pltpu.SemaphoreType.DMA,
            pltpu.SemaphoreType.DMA,
            pltpu.SemaphoreType.DMA,
            pltpu.SemaphoreType.DMA,
            pltpu.SemaphoreType.DMA,
            pltpu.SemaphoreType.DMA,
            pltpu.SemaphoreType.DMA,
        ],
    )
    def k(x_hbm, comb_hbm, out_hbm, comb_v, rows0, rows1, acc_sh,
          si0, si1, si2, si3, sg0, sg1, ss0, ss1):
        c = lax.axis_index("c")
        s = lax.axis_index("s")
        wid = s * NC + c
        lo = wid * epb
        isems = (si0, si1, si2, si3)
        bufs = (rows0, rows1)
        gsems = (sg0, sg1)
        ssems = (ss0, ss1)

        def idx_start(t, q):
            pltpu.async_copy(comb_hbm.at[pl.ds((lo + t) * 3, 3)],
                             comb_v.at[pl.ds(q * 3, 3)], isems[q])

        def idx_wait(t, q):
            pltpu.make_async_copy(comb_hbm.at[pl.ds((lo + t) * 3, 3)],
                                  comb_v.at[pl.ds(q * 3, 3)], isems[q]).wait()

        def gather_start(q, p):
            pltpu.async_copy(x_hbm.at[comb_v.at[q * 3, 0]], bufs[p], gsems[p])

        def gather_wait(q, p):
            pltpu.make_async_copy(
                x_hbm.at[comb_v.at[q * 3, 0]], bufs[p], gsems[p]).wait()

        def scale(q, p):
            buf = bufs[p]

            @pl.loop(0, B, step=4)
            def _(i):
                row = jnp.full((LANES,), q * 3 + 2, jnp.int32)
                col = jnp.full((LANES,), 0, jnp.int32)
                wbs = [plsc.bitcast(
                    plsc.load_gather(
                        comb_v, [row, col,
                                 jnp.full((LANES,), i + u, jnp.int32)]),
                    jnp.float32) for u in range(4)]
                for u in range(4):
                    for jj in range(d // LANES):
                        sl = pl.ds(jj * LANES, LANES)
                        buf[i + u, sl] = buf[i + u, sl] * wbs[u]

        def scatter_start(q, p):
            pltpu.async_copy(bufs[p], acc_sh.at[comb_v.at[q * 3 + 1, 0]],
                             ssems[p], add=True)

        def scatter_wait(q, p):
            pltpu.make_async_copy(bufs[p], acc_sh.at[comb_v.at[q * 3 + 1, 0]],
                                  ssems[p]).wait()

        for t in range(3):
            idx_start(t, t)

        zero = jnp.zeros((LANES,), jnp.float32)

        @pl.loop(0, ZC)
        def _(i):
            for j in range(d // LANES):
                rows0[i, pl.ds(j * LANES, LANES)] = zero

        @pl.loop(s, nchunk, step=NS)
        def _(t):
            pltpu.sync_copy(rows0.at[pl.ds(0, ZC)],
                            acc_sh.at[pl.ds(t * ZC, ZC)])

        idx_wait(0, 0)
        gather_start(0, 0)
        plsc.subcore_barrier()

        @pl.loop(0, epb, step=NQ)
        def _(j):
            for b in range(NQ):
                t = j + b
                q, p = b, b % 2
                qn, pn = (b + 1) % NQ, (b + 1) % 2
                qprev = (b + 3) % NQ

                @pl.when(t + 1 < epb)
                def _():
                    idx_wait(t + 1, qn)

                gather_wait(q, p)

                @pl.when(t > 0)
                def _():
                    scatter_wait(qprev, pn)

                @pl.when(t + 1 < epb)
                def _():
                    gather_start(qn, pn)

                scale(q, p)
                scatter_start(q, p)

                @pl.when(t + 3 < epb)
                def _():
                    idx_start(t + 3, (b + 3) % NQ)

        scatter_wait((epb - 1) % NQ, (epb - 1) % 2)
        plsc.subcore_barrier()

        @pl.loop(s, nchunk, step=NS)
        def _(t):
            pltpu.sync_copy(acc_sh.at[pl.ds(t * ZC, ZC)],
                            out_hbm.at[c].at[pl.ds(t * ZC, ZC)])

    return k(x, comb)


def _tc_finish(y, w):
    _, n, d = y.shape
    blk = 2000

    def body(y_ref, w_ref, o_ref):
        acc = y_ref[0] + y_ref[1]
        o_ref[...] = jnp.maximum(
            jnp.dot(acc, w_ref[...], preferred_element_type=jnp.float32), 0.0)

    return pl.pallas_call(
        body,
        out_shape=jax.ShapeDtypeStruct((n, d), jnp.float32),
        grid=(n // blk,),
        in_specs=[
            pl.BlockSpec((2, blk, d), lambda i: (0, i, 0)),
            pl.BlockSpec((d, d), lambda i: (0, 0)),
        ],
        out_specs=pl.BlockSpec((blk, d), lambda i: (i, 0)),
    )(y, w)


def kernel(x, edge_index, edge_weight, W):
    e = edge_index.shape[1]
    nw = NC * NS
    epb = -(-e // (nw * B))
    epb = -(-epb // NQ) * NQ
    ep = nw * epb * B
    pad = ep - e
    pad_idx = (jnp.arange(pad, dtype=jnp.int32) %
               jnp.int32(x.shape[0])).astype(jnp.int32)
    src = jnp.concatenate([edge_index[0], pad_idx]).reshape(-1, 1, B)
    dst = jnp.concatenate([edge_index[1], pad_idx]).reshape(-1, 1, B)
    wbits = jax.lax.bitcast_convert_type(
        jnp.concatenate([edge_weight, jnp.zeros((pad,), jnp.float32)]),
        jnp.int32).reshape(-1, 1, B)
    comb = jnp.concatenate([src, dst, wbits],
                           axis=1).reshape(-1, 1, B)
    y = _sc_spmm(x, comb, epb)
    return _tc_finish(y, W)


# speedup vs baseline: 1.2001x; 1.0005x over previous
import dataclasses
import functools

import jax
import jax.numpy as jnp
from jax import lax
from jax.experimental import pallas as pl
from jax.experimental.pallas import tpu as pltpu
from jax.experimental.pallas import tpu_sc as plsc

NC = 2
NS = 16
LANES = 16
B = 128
ZC = 80
NQ = 4


def _sc_spmm(x, comb, epb):
    n, d = x.shape
    nchunk = n // ZC
    mesh = plsc.VectorSubcoreMesh(core_axis_name="c", subcore_axis_name="s")
    cp = pltpu.CompilerParams()
    if "needs_layout_passes" in pltpu.CompilerParams.__dataclass_fields__:
        cp = dataclasses.replace(cp, needs_layout_passes=False)

    @functools.partial(
        pl.kernel,
        out_type=jax.ShapeDtypeStruct((NC, n, d), jnp.float32),
        mesh=mesh,
        compiler_params=cp,
        scratch_types=[
            pltpu.VMEM((NQ * 3, 1, B), jnp.int32),
            pltpu.VMEM((B, d), jnp.float32),
            pltpu.VMEM((B, d), jnp.float32),
            pltpu.VMEM_SHARED((n, d), jnp.float32),
            pltpu.SemaphoreType.DMA,
            pltpu.SemaphoreType.DMA,
            pltpu.SemaphoreType.DMA,
            pltpu.SemaphoreType.DMA,
            pltpu.SemaphoreType.DMA,
            pltpu.SemaphoreType.DMA,
            pltpu.SemaphoreType.DMA,
            pltpu.SemaphoreType.DMA,
            pltpu.SemaphoreType.DMA,
        ],
    )
    def k(x_hbm, comb_hbm, out_hbm, comb_v, rows0, rows1, acc_sh,
          si0, si1, si2, si3, sg0, sg1, ss0, ss1, sch):
        c = lax.axis_index("c")
        s = lax.axis_index("s")
        wid = s * NC + c
        lo = wid * epb
        isems = (si0, si1, si2, si3)
        bufs = (rows0, rows1)
        gsems = (sg0, sg1)
        ssems = (ss0, ss1)

        def idx_start(t, q):
            pltpu.async_copy(comb_hbm.at[pl.ds((lo + t) * 3, 3)],
                             comb_v.at[pl.ds(q * 3, 3)], isems[q])

        def idx_wait(t, q):
            pltpu.make_async_copy(comb_hbm.at[pl.ds((lo + t) * 3, 3)],
                                  comb_v.at[pl.ds(q * 3, 3)], isems[q]).wait()

        def gather_start(q, p):
            pltpu.async_copy(x_hbm.at[comb_v.at[q * 3, 0]], bufs[p], gsems[p])

        def gather_wait(q, p):
            pltpu.make_async_copy(
                x_hbm.at[comb_v.at[q * 3, 0]], bufs[p], gsems[p]).wait()

        def scale(q, p):
            buf = bufs[p]

            @pl.loop(0, B, step=4)
            def _(i):
                row = jnp.full((LANES,), q * 3 + 2, jnp.int32)
                col = jnp.full((LANES,), 0, jnp.int32)
                wbs = [plsc.bitcast(
                    plsc.load_gather(
                        comb_v, [row, col,
                                 jnp.full((LANES,), i + u, jnp.int32)]),
                    jnp.float32) for u in range(4)]
                for u in range(4):
                    for jj in range(d // LANES):
                        sl = pl.ds(jj * LANES, LANES)
                        buf[i + u, sl] = buf[i + u, sl] * wbs[u]

        def scatter_start(q, p):
            pltpu.async_copy(bufs[p], acc_sh.at[comb_v.at[q * 3 + 1, 0]],
                             ssems[p], add=True)

        def scatter_wait(q, p):
            pltpu.make_async_copy(bufs[p], acc_sh.at[comb_v.at[q * 3 + 1, 0]],
                                  ssems[p]).wait()

        for t in range(3):
            idx_start(t, t)

        zero = jnp.zeros((LANES,), jnp.float32)

        @pl.loop(0, ZC)
        def _(i):
            for j in range(d // LANES):
                rows0[i, pl.ds(j * LANES, LANES)] = zero

        @pl.loop(s, nchunk, step=NS)
        def _(t):
            pltpu.async_copy(rows0.at[pl.ds(0, ZC)],
                             acc_sh.at[pl.ds(t * ZC, ZC)], sch)

        @pl.loop(s, nchunk, step=NS)
        def _(t):
            pltpu.make_async_copy(rows0.at[pl.ds(0, ZC)],
                                  acc_sh.at[pl.ds(t * ZC, ZC)], sch).wait()

        idx_wait(0, 0)
        gather_start(0, 0)
        plsc.subcore_barrier()

        @pl.loop(0, epb, step=NQ)
        def _(j):
            for b in range(NQ):
                t = j + b
                q, p = b, b % 2
                qn, pn = (b + 1) % NQ, (b + 1) % 2
                qprev = (b + 3) % NQ

                @pl.when(t + 1 < epb)
                def _():
                    idx_wait(t + 1, qn)

                gather_wait(q, p)

                @pl.when(t > 0)
                def _():
                    scatter_wait(qprev, pn)

                @pl.when(t + 1 < epb)
                def _():
                    gather_start(qn, pn)

                scale(q, p)
                scatter_start(q, p)

                @pl.when(t + 3 < epb)
                def _():
                    idx_start(t + 3, (b + 3) % NQ)

        scatter_wait((epb - 1) % NQ, (epb - 1) % 2)
        plsc.subcore_barrier()

        @pl.loop(s, nchunk, step=NS)
        def _(t):
            pltpu.async_copy(acc_sh.at[pl.ds(t * ZC, ZC)],
                             out_hbm.at[c].at[pl.ds(t * ZC, ZC)], sch)

        @pl.loop(s, nchunk, step=NS)
        def _(t):
            pltpu.make_async_copy(acc_sh.at[pl.ds(t * ZC, ZC)],
                                  out_hbm.at[c].at[pl.ds(t * ZC, ZC)],
                                  sch).wait()

    return k(x, comb)


def _tc_finish(y, w):
    _, n, d = y.shape
    blk = 2000

    def body(y_ref, w_ref, o_ref):
        acc = y_ref[0] + y_ref[1]
        o_ref[...] = jnp.maximum(
            jnp.dot(acc, w_ref[...], preferred_element_type=jnp.float32), 0.0)

    return pl.pallas_call(
        body,
        out_shape=jax.ShapeDtypeStruct((n, d), jnp.float32),
        grid=(n // blk,),
        in_specs=[
            pl.BlockSpec((2, blk, d), lambda i: (0, i, 0)),
            pl.BlockSpec((d, d), lambda i: (0, 0)),
        ],
        out_specs=pl.BlockSpec((blk, d), lambda i: (i, 0)),
    )(y, w)


def kernel(x, edge_index, edge_weight, W):
    e = edge_index.shape[1]
    nw = NC * NS
    epb = -(-e // (nw * B))
    epb = -(-epb // NQ) * NQ
    ep = nw * epb * B
    pad = ep - e
    pad_idx = (jnp.arange(pad, dtype=jnp.int32) %
               jnp.int32(x.shape[0])).astype(jnp.int32)
    src = jnp.concatenate([edge_index[0], pad_idx]).reshape(-1, 1, B)
    dst = jnp.concatenate([edge_index[1], pad_idx]).reshape(-1, 1, B)
    wbits = jax.lax.bitcast_convert_type(
        jnp.concatenate([edge_weight, jnp.zeros((pad,), jnp.float32)]),
        jnp.int32).reshape(-1, 1, B)
    comb = jnp.concatenate([src, dst, wbits],
                           axis=1).reshape(-1, 1, B)
    y = _sc_spmm(x, comb, epb)
    return _tc_finish(y, W)
